# baseline (speedup 1.0000x reference)
import functools

import jax
import jax.numpy as jnp
from jax import lax
from jax.experimental import pallas as pl
from jax.experimental.pallas import tpu as pltpu
from jax.experimental.pallas import tpu_sc as plsc

N = 10000
E = 320000
DIN = 128
H = 64
G = 64

NC = 2
NS = 16
CHUNK = 128
NCHUNKS = E // CHUNK
CPC = NCHUNKS // NC
BASE_TRIPS = CPC // NS
EXTRA = CPC - BASE_TRIPS * NS
K = 4
NW = NC * NS
NBLOCKS = NCHUNKS // K
BASE_BLOCKS = NBLOCKS // NW
EXTRA_BLOCKS = NBLOCKS - BASE_BLOCKS * NW
MAX_BLOCKS = BASE_BLOCKS + 1
SLAB = 128
DK = 4
DNBLOCKS = NCHUNKS // DK
DBASE = DNBLOCKS // NW
DEXTRA = DNBLOCKS - DBASE * NW
DMAX = DBASE + 1
SLICE = 640
LAST_SLICE = N - (NS - 1) * SLICE

_mesh = plsc.VectorSubcoreMesh(core_axis_name="c", subcore_axis_name="s")


def _sc_deg(col2d, ones128, zcol):
    def body(col_hbm, ones_hbm, z_hbm, deg0_hbm, deg1_hbm, idx_a, ss_a,
             idx_b, ss_b, ones_v, slab_v, acc_sh):
        cid = lax.axis_index("c")
        sid = lax.axis_index("s")
        pltpu.sync_copy(ones_hbm, ones_v)
        pltpu.sync_copy(z_hbm, slab_v)

        @pl.when(sid < NS - 1)
        def _():
            pltpu.sync_copy(slab_v, acc_sh.at[pl.ds(sid * SLICE, SLICE)])

        @pl.when(sid == NS - 1)
        def _():
            pltpu.sync_copy(slab_v.at[pl.ds(0, LAST_SLICE)],
                            acc_sh.at[pl.ds(sid * SLICE, LAST_SLICE)])

        plsc.subcore_barrier()
        wid = cid * NS + sid
        ntrips = jnp.where(wid < DEXTRA, DBASE + 1, DBASE)
        bufs = ((idx_a, ss_a), (idx_b, ss_b))

        def load(t, idx_vv):
            bid = (wid + t * NW) * DK
            pltpu.sync_copy(col_hbm.at[pl.ds(bid, DK)], idx_vv)

        def drain_sc(idx_vv, ssem):
            for j in range(DK):
                pltpu.make_async_copy(ones_v, acc_sh.at[idx_vv.at[j]],
                                      ssem).wait()

        @pl.when(ntrips > 0)
        def _():
            load(0, bufs[0][0])

        def pair(p, c):
            for half in (0, 1):
                t = 2 * p + half

                @pl.when(t < ntrips)
                def _(t=t, half=half):
                    idx_vv, ssem = bufs[half]
                    for j in range(DK):
                        pltpu.async_copy(ones_v, acc_sh.at[idx_vv.at[j]],
                                         ssem, add=True)

                    @pl.when(t + 1 < ntrips)
                    def _(t=t, half=half):
                        @pl.when(t >= 1)
                        def _(half=half):
                            drain_sc(*bufs[1 - half])
                        load(t + 1, bufs[1 - half][0])
            return c

        lax.fori_loop(0, (DMAX + 1) // 2, pair, 0)
        drain_sc(*bufs[0])
        drain_sc(*bufs[1])
        plsc.subcore_barrier()
        for c, dst in ((0, deg0_hbm), (1, deg1_hbm)):
            @pl.when(cid == c)
            def _(dst=dst):
                @pl.when(sid < NS - 1)
                def _():
                    pltpu.sync_copy(acc_sh.at[pl.ds(sid * SLICE, SLICE)],
                                    slab_v)
                    pltpu.sync_copy(slab_v, dst.at[pl.ds(sid * SLICE, SLICE)])

                @pl.when(sid == NS - 1)
                def _():
                    pltpu.sync_copy(acc_sh.at[pl.ds(sid * SLICE, LAST_SLICE)],
                                    slab_v.at[pl.ds(0, LAST_SLICE)])
                    pltpu.sync_copy(slab_v.at[pl.ds(0, LAST_SLICE)],
                                    dst.at[pl.ds(sid * SLICE, LAST_SLICE)])

    f = pl.kernel(
        body,
        out_type=[jax.ShapeDtypeStruct((N,), jnp.float32),
                  jax.ShapeDtypeStruct((N,), jnp.float32)],
        mesh=_mesh,
        compiler_params=pltpu.CompilerParams(use_tc_tiling_on_sc=False),
        scratch_types=[
            pltpu.VMEM((DK, CHUNK), jnp.int32),
            pltpu.SemaphoreType.DMA,
            pltpu.VMEM((DK, CHUNK), jnp.int32),
            pltpu.SemaphoreType.DMA,
            pltpu.VMEM((CHUNK,), jnp.float32),
            pltpu.VMEM((SLICE,), jnp.float32),
            pltpu.VMEM_SHARED((N,), jnp.float32),
        ],
    )
    return f(col2d, ones128, zcol)


def _sc_agg(y, ec, zslab):
    def body(y_hbm, ec_hbm, z_hbm, u0_hbm, u1_hbm,
             eidx_a, rows_a, gs_a,
             eidx_b, rows_b, gs_b,
             slab_v, acc_sh):
        cid = lax.axis_index("c")
        sid = lax.axis_index("s")
        wid = cid * NS + sid
        pltpu.sync_copy(z_hbm, slab_v)

        @pl.when(sid < NS - 1)
        def _():
            for i in range(SLICE // SLAB):
                pltpu.sync_copy(slab_v,
                                acc_sh.at[pl.ds(sid * SLICE + i * SLAB, SLAB)])

        @pl.when(sid == NS - 1)
        def _():
            for i in range(LAST_SLICE // SLAB):
                pltpu.sync_copy(slab_v,
                                acc_sh.at[pl.ds(sid * SLICE + i * SLAB, SLAB)])
            rem = LAST_SLICE % SLAB
            if rem:
                pltpu.sync_copy(
                    slab_v.at[pl.ds(0, rem)],
                    acc_sh.at[pl.ds(sid * SLICE + LAST_SLICE - rem, rem)])

        plsc.subcore_barrier()
        ntrips = jnp.where(wid < EXTRA_BLOCKS, BASE_BLOCKS + 1, BASE_BLOCKS)

        bufs = ((eidx_a, rows_a, gs_a), (eidx_b, rows_b, gs_b))

        def fire(t, buf):
            eidx_v, rows_v, gsem = buf
            bid = (wid + t * NW) * K
            pltpu.sync_copy(ec_hbm.at[pl.ds(bid, K)], eidx_v)
            for j in range(K):
                pltpu.async_copy(y_hbm.at[eidx_v.at[j, 0]], rows_v.at[j], gsem)

        def drain_gathers(buf):
            _, rows_v, gsem = buf
            for j in range(K):
                pltpu.make_async_copy(y_hbm.at[pl.ds(0, CHUNK)],
                                      rows_v.at[j], gsem).wait()

        def sync_scatters(buf):
            eidx_v, rows_v, _ = buf
            for j in range(K):
                pltpu.sync_copy(rows_v.at[j], acc_sh.at[eidx_v.at[j, 1]],
                                add=True)

        @pl.when(ntrips > 0)
        def _():
            fire(0, bufs[0])

        def pair(p, c):
            for half in (0, 1):
                t = 2 * p + half

                @pl.when(t < ntrips)
                def _(t=t, half=half):
                    @pl.when(t + 1 < ntrips)
                    def _(t=t, half=half):
                        fire(t + 1, bufs[1 - half])
                    drain_gathers(bufs[half])
                    sync_scatters(bufs[half])
            return c

        lax.fori_loop(0, (MAX_BLOCKS + 1) // 2, pair, 0)
        plsc.subcore_barrier()

        def put(dst, off, nrows):
            pltpu.sync_copy(acc_sh.at[pl.ds(off, nrows)],
                            slab_v.at[pl.ds(0, nrows)])
            pltpu.sync_copy(slab_v.at[pl.ds(0, nrows)],
                            dst.at[pl.ds(off, nrows)])

        for c, dst in ((0, u0_hbm), (1, u1_hbm)):
            @pl.when(cid == c)
            def _(dst=dst):
                @pl.when(sid < NS - 1)
                def _():
                    for i in range(SLICE // SLAB):
                        put(dst, sid * SLICE + i * SLAB, SLAB)

                @pl.when(sid == NS - 1)
                def _():
                    for i in range(LAST_SLICE // SLAB):
                        put(dst, sid * SLICE + i * SLAB, SLAB)
                    rem = LAST_SLICE % SLAB
                    if rem:
                        put(dst, sid * SLICE + LAST_SLICE - rem, rem)

    f = pl.kernel(
        body,
        out_type=[jax.ShapeDtypeStruct((N, H), jnp.float32),
                  jax.ShapeDtypeStruct((N, H), jnp.float32)],
        mesh=_mesh,
        compiler_params=pltpu.CompilerParams(use_tc_tiling_on_sc=False),
        scratch_types=[
            pltpu.VMEM((K, 2, CHUNK), jnp.int32),
            pltpu.VMEM((K, CHUNK, H), jnp.float32),
            pltpu.SemaphoreType.DMA,
            pltpu.VMEM((K, 2, CHUNK), jnp.int32),
            pltpu.VMEM((K, CHUNK, H), jnp.float32),
            pltpu.SemaphoreType.DMA,
            pltpu.VMEM((SLAB, H), jnp.float32),
            pltpu.VMEM_SHARED((N, H), jnp.float32),
        ],
    )
    return f(y, ec, zslab)


def _tc_first(x, W1, d0, d1):
    def body(x_ref, w_ref, d0_ref, d1_ref, xw_ref, y_ref):
        deg = d0_ref[...] + d1_ref[...] + 1.0
        dinv = lax.rsqrt(deg)
        xw = jnp.dot(x_ref[...], w_ref[...],
                     preferred_element_type=jnp.float32,
                     precision=lax.Precision.HIGHEST)
        xw_ref[...] = xw
        y_ref[...] = xw * dinv

    return pl.pallas_call(
        body,
        compiler_params=pltpu.CompilerParams(vmem_limit_bytes=100 * 1024 * 1024),
        out_shape=[
            jax.ShapeDtypeStruct((N, H), jnp.float32),
            jax.ShapeDtypeStruct((N, H), jnp.float32),
        ],
    )(x, W1, d0, d1)


def _tc_mid(U0, U1, xw1, d0, d1, b1, W2):
    def body(u0_ref, u1_ref, xw1_ref, d0_ref, d1_ref, b_ref, w_ref,
             xw2_ref, y2_ref):
        deg = d0_ref[...] + d1_ref[...] + 1.0
        dinv = lax.rsqrt(deg)
        h = jnp.maximum(
            dinv * (u0_ref[...] + u1_ref[...]) + dinv * dinv * xw1_ref[...]
            + b_ref[...], 0.0)
        xw2 = jnp.dot(h, w_ref[...], preferred_element_type=jnp.float32, precision=lax.Precision.HIGHEST)
        xw2_ref[...] = xw2
        y2_ref[...] = xw2 * dinv

    return pl.pallas_call(
        body,
        compiler_params=pltpu.CompilerParams(vmem_limit_bytes=100 * 1024 * 1024),
        out_shape=[
            jax.ShapeDtypeStruct((N, H), jnp.float32),
            jax.ShapeDtypeStruct((N, H), jnp.float32),
        ],
    )(U0, U1, xw1, d0, d1, b1, W2)


def _tc_last(U0, U1, xw2, d0, d1, b2, batch2, Wf1, bf1, Wf2, bf2):
    def body(u0_ref, u1_ref, xw2_ref, d0_ref, d1_ref, b_ref, bat_ref,
             wf1_ref, bf1_ref, wf2_ref, bf2_ref, out_ref):
        deg = d0_ref[...] + d1_ref[...] + 1.0
        dinv = lax.rsqrt(deg)
        h = jnp.maximum(
            dinv * (u0_ref[...] + u1_ref[...]) + dinv * dinv * xw2_ref[...]
            + b_ref[...], 0.0)
        gids = lax.broadcasted_iota(jnp.int32, (1, G), 1)
        onehot = (bat_ref[...] == gids).astype(jnp.float32)
        dn = (((0,), (0,)), ((), ()))
        sums = lax.dot_general(onehot, h, dn, preferred_element_type=jnp.float32, precision=lax.Precision.HIGHEST)
        cnts = lax.dot_general(onehot, jnp.ones((N, 1), jnp.float32), dn,
                               preferred_element_type=jnp.float32, precision=lax.Precision.HIGHEST)
        p = sums / jnp.maximum(cnts, 1.0)
        q = jnp.maximum(
            jnp.dot(p, wf1_ref[...], preferred_element_type=jnp.float32, precision=lax.Precision.HIGHEST)
            + bf1_ref[...], 0.0)
        out_ref[...] = (
            jnp.dot(q, wf2_ref[...], preferred_element_type=jnp.float32, precision=lax.Precision.HIGHEST)
            + bf2_ref[...])

    return pl.pallas_call(
        body,
        compiler_params=pltpu.CompilerParams(vmem_limit_bytes=100 * 1024 * 1024),
        out_shape=jax.ShapeDtypeStruct((G, 1), jnp.float32),
    )(U0, U1, xw2, d0, d1, b2, batch2, Wf1, bf1, Wf2, bf2)


def kernel(x, edge_index, edge_attr, batch, W1, b1, W2, b2, Wf1, bf1, Wf2, bf2):
    del edge_attr
    row = edge_index[0]
    col = edge_index[1]
    ones128 = jnp.ones((CHUNK,), jnp.float32)
    zslab = jnp.zeros((SLAB, H), jnp.float32)
    zcol = jnp.zeros((SLICE,), jnp.float32)

    dp0, dp1 = _sc_deg(col.reshape(NCHUNKS, CHUNK), ones128, zcol)
    d0 = dp0.reshape(N, 1)
    d1 = dp1.reshape(N, 1)

    xw1, y1 = _tc_first(x, W1, d0, d1)
    ec = jnp.stack([row.reshape(NCHUNKS, CHUNK), col.reshape(NCHUNKS, CHUNK)],
                   axis=1)
    U10, U11 = _sc_agg(y1, ec, zslab)
    xw2, y2 = _tc_mid(U10, U11, xw1, d0, d1, b1.reshape(1, H), W2)
    U20, U21 = _sc_agg(y2, ec, zslab)
    out = _tc_last(U20, U21, xw2, d0, d1, b2.reshape(1, H),
                   batch.reshape(N, 1), Wf1, bf1.reshape(1, H // 2),
                   Wf2, bf2.reshape(1, 1))
    return out


# speedup vs baseline: 1.0221x; 1.0221x over previous
import functools

import jax
import jax.numpy as jnp
from jax import lax
from jax.experimental import pallas as pl
from jax.experimental.pallas import tpu as pltpu
from jax.experimental.pallas import tpu_sc as plsc

N = 10000
E = 320000
DIN = 128
H = 64
G = 64

NC = 2
NS = 16
CHUNK = 128
NCHUNKS = E // CHUNK
CPC = NCHUNKS // NC
BASE_TRIPS = CPC // NS
EXTRA = CPC - BASE_TRIPS * NS
K = 4
NW = NC * NS
NBLOCKS = NCHUNKS // K
BASE_BLOCKS = NBLOCKS // NW
EXTRA_BLOCKS = NBLOCKS - BASE_BLOCKS * NW
MAX_BLOCKS = BASE_BLOCKS + 1
SLAB = 128
DK = 4
DNBLOCKS = NCHUNKS // DK
DBASE = DNBLOCKS // NW
DEXTRA = DNBLOCKS - DBASE * NW
DMAX = DBASE + 1
SLICE = 640
LAST_SLICE = N - (NS - 1) * SLICE

_mesh = plsc.VectorSubcoreMesh(core_axis_name="c", subcore_axis_name="s")


def _sc_deg(col2d, ones128, zcol):
    def body(col_hbm, ones_hbm, z_hbm, deg0_hbm, deg1_hbm, idx_a, ss_a,
             idx_b, ss_b, ones_v, slab_v, acc_sh):
        cid = lax.axis_index("c")
        sid = lax.axis_index("s")
        pltpu.sync_copy(ones_hbm, ones_v)
        pltpu.sync_copy(z_hbm, slab_v)

        @pl.when(sid < NS - 1)
        def _():
            pltpu.sync_copy(slab_v, acc_sh.at[pl.ds(sid * SLICE, SLICE)])

        @pl.when(sid == NS - 1)
        def _():
            pltpu.sync_copy(slab_v.at[pl.ds(0, LAST_SLICE)],
                            acc_sh.at[pl.ds(sid * SLICE, LAST_SLICE)])

        plsc.subcore_barrier()
        wid = cid * NS + sid
        ntrips = jnp.where(wid < DEXTRA, DBASE + 1, DBASE)
        bufs = ((idx_a, ss_a), (idx_b, ss_b))

        def load(t, idx_vv):
            bid = (wid + t * NW) * DK
            pltpu.sync_copy(col_hbm.at[pl.ds(bid, DK)], idx_vv)

        def drain_sc(idx_vv, ssem):
            for j in range(DK):
                pltpu.make_async_copy(ones_v, acc_sh.at[idx_vv.at[j]],
                                      ssem).wait()

        @pl.when(ntrips > 0)
        def _():
            load(0, bufs[0][0])

        def pair(p, c):
            for half in (0, 1):
                t = 2 * p + half

                @pl.when(t < ntrips)
                def _(t=t, half=half):
                    idx_vv, ssem = bufs[half]
                    for j in range(DK):
                        pltpu.async_copy(ones_v, acc_sh.at[idx_vv.at[j]],
                                         ssem, add=True)

                    @pl.when(t + 1 < ntrips)
                    def _(t=t, half=half):
                        @pl.when(t >= 1)
                        def _(half=half):
                            drain_sc(*bufs[1 - half])
                        load(t + 1, bufs[1 - half][0])
            return c

        lax.fori_loop(0, (DMAX + 1) // 2, pair, 0)
        drain_sc(*bufs[0])
        drain_sc(*bufs[1])
        plsc.subcore_barrier()
        for c, dst in ((0, deg0_hbm), (1, deg1_hbm)):
            @pl.when(cid == c)
            def _(dst=dst):
                @pl.when(sid < NS - 1)
                def _():
                    pltpu.sync_copy(acc_sh.at[pl.ds(sid * SLICE, SLICE)],
                                    slab_v)
                    pltpu.sync_copy(slab_v, dst.at[pl.ds(sid * SLICE, SLICE)])

                @pl.when(sid == NS - 1)
                def _():
                    pltpu.sync_copy(acc_sh.at[pl.ds(sid * SLICE, LAST_SLICE)],
                                    slab_v.at[pl.ds(0, LAST_SLICE)])
                    pltpu.sync_copy(slab_v.at[pl.ds(0, LAST_SLICE)],
                                    dst.at[pl.ds(sid * SLICE, LAST_SLICE)])

    f = pl.kernel(
        body,
        out_type=[jax.ShapeDtypeStruct((N,), jnp.float32),
                  jax.ShapeDtypeStruct((N,), jnp.float32)],
        mesh=_mesh,
        compiler_params=pltpu.CompilerParams(use_tc_tiling_on_sc=False),
        scratch_types=[
            pltpu.VMEM((DK, CHUNK), jnp.int32),
            pltpu.SemaphoreType.DMA,
            pltpu.VMEM((DK, CHUNK), jnp.int32),
            pltpu.SemaphoreType.DMA,
            pltpu.VMEM((CHUNK,), jnp.float32),
            pltpu.VMEM((SLICE,), jnp.float32),
            pltpu.VMEM_SHARED((N,), jnp.float32),
        ],
    )
    return f(col2d, ones128, zcol)


def _sc_agg(y, ec, zslab):
    def body(y_hbm, ec_hbm, z_hbm, u0_hbm, u1_hbm,
             eidx_a, rows_a, gs_a,
             eidx_b, rows_b, gs_b,
             slab_v, slab2_v, ws, acc_sh):
        cid = lax.axis_index("c")
        sid = lax.axis_index("s")
        wid = cid * NS + sid
        pltpu.sync_copy(z_hbm, slab_v)

        def my_chunks():
            if_last = [(i * SLAB, SLAB) for i in range(LAST_SLICE // SLAB)]
            rem = LAST_SLICE % SLAB
            if rem:
                if_last.append((LAST_SLICE - rem, rem))
            return [(i * SLAB, SLAB) for i in range(SLICE // SLAB)], if_last

        full_chunks, last_chunks = my_chunks()

        def zero_init(chunks):
            zs = []
            for off, n in chunks:
                dst = acc_sh.at[pl.ds(sid * SLICE + off, n)]
                src = slab_v.at[pl.ds(0, n)]
                pltpu.async_copy(src, dst, ws)
                zs.append((src, dst))
            for src, dst in zs:
                pltpu.make_async_copy(src, dst, ws).wait()

        @pl.when(sid < NS - 1)
        def _():
            zero_init(full_chunks)

        @pl.when(sid == NS - 1)
        def _():
            zero_init(last_chunks)

        plsc.subcore_barrier()
        ntrips = jnp.where(wid < EXTRA_BLOCKS, BASE_BLOCKS + 1, BASE_BLOCKS)

        bufs = ((eidx_a, rows_a, gs_a), (eidx_b, rows_b, gs_b))

        def fire(t, buf):
            eidx_v, rows_v, gsem = buf
            bid = (wid + t * NW) * K
            pltpu.sync_copy(ec_hbm.at[pl.ds(bid, K)], eidx_v)
            for j in range(K):
                pltpu.async_copy(y_hbm.at[eidx_v.at[j, 0]], rows_v.at[j], gsem)

        def drain_gathers(buf):
            _, rows_v, gsem = buf
            for j in range(K):
                pltpu.make_async_copy(y_hbm.at[pl.ds(0, CHUNK)],
                                      rows_v.at[j], gsem).wait()

        def sync_scatters(buf):
            eidx_v, rows_v, _ = buf
            for j in range(K):
                pltpu.sync_copy(rows_v.at[j], acc_sh.at[eidx_v.at[j, 1]],
                                add=True)

        @pl.when(ntrips > 0)
        def _():
            fire(0, bufs[0])

        def pair(p, c):
            for half in (0, 1):
                t = 2 * p + half

                @pl.when(t < ntrips)
                def _(t=t, half=half):
                    @pl.when(t + 1 < ntrips)
                    def _(t=t, half=half):
                        fire(t + 1, bufs[1 - half])
                    drain_gathers(bufs[half])
                    sync_scatters(bufs[half])
            return c

        lax.fori_loop(0, (MAX_BLOCKS + 1) // 2, pair, 0)
        plsc.subcore_barrier()

        def writeout(dst, chunks):
            slabs = (slab_v, slab2_v)
            pushes = []
            for i, (off, n) in enumerate(chunks):
                sl = slabs[i % 2].at[pl.ds(0, n)]
                if i >= 2:
                    pltpu.make_async_copy(*pushes[i - 2], ws).wait()
                pltpu.sync_copy(acc_sh.at[pl.ds(sid * SLICE + off, n)], sl)
                d = dst.at[pl.ds(sid * SLICE + off, n)]
                pltpu.async_copy(sl, d, ws)
                pushes.append((sl, d))
            for src, d in pushes[-2:]:
                pltpu.make_async_copy(src, d, ws).wait()

        for c, dst in ((0, u0_hbm), (1, u1_hbm)):
            @pl.when(cid == c)
            def _(dst=dst):
                @pl.when(sid < NS - 1)
                def _():
                    writeout(dst, full_chunks)

                @pl.when(sid == NS - 1)
                def _():
                    writeout(dst, last_chunks)

    f = pl.kernel(
        body,
        out_type=[jax.ShapeDtypeStruct((N, H), jnp.float32),
                  jax.ShapeDtypeStruct((N, H), jnp.float32)],
        mesh=_mesh,
        compiler_params=pltpu.CompilerParams(use_tc_tiling_on_sc=False),
        scratch_types=[
            pltpu.VMEM((K, 2, CHUNK), jnp.int32),
            pltpu.VMEM((K, CHUNK, H), jnp.float32),
            pltpu.SemaphoreType.DMA,
            pltpu.VMEM((K, 2, CHUNK), jnp.int32),
            pltpu.VMEM((K, CHUNK, H), jnp.float32),
            pltpu.SemaphoreType.DMA,
            pltpu.VMEM((SLAB, H), jnp.float32),
            pltpu.VMEM((SLAB, H), jnp.float32),
            pltpu.SemaphoreType.DMA,
            pltpu.VMEM_SHARED((N, H), jnp.float32),
        ],
    )
    return f(y, ec, zslab)


def _tc_mm(x, W1):
    def body(x_ref, w_ref, xw_ref):
        xw_ref[...] = jnp.dot(x_ref[...], w_ref[...],
                              preferred_element_type=jnp.float32,
                              precision=lax.Precision.HIGHEST)

    return pl.pallas_call(
        body,
        compiler_params=pltpu.CompilerParams(vmem_limit_bytes=100 * 1024 * 1024),
        out_shape=jax.ShapeDtypeStruct((N, H), jnp.float32),
    )(x, W1)


def _tc_scale(xw, d0, d1):
    def body(xw_ref, d0_ref, d1_ref, y_ref):
        deg = d0_ref[...] + d1_ref[...] + 1.0
        dinv = lax.rsqrt(deg)
        y_ref[...] = xw_ref[...] * dinv

    return pl.pallas_call(
        body,
        compiler_params=pltpu.CompilerParams(vmem_limit_bytes=100 * 1024 * 1024),
        out_shape=jax.ShapeDtypeStruct((N, H), jnp.float32),
    )(xw, d0, d1)


def _tc_mid(U0, U1, xw1, d0, d1, b1, W2):
    def body(u0_ref, u1_ref, xw1_ref, d0_ref, d1_ref, b_ref, w_ref,
             xw2_ref, y2_ref):
        deg = d0_ref[...] + d1_ref[...] + 1.0
        dinv = lax.rsqrt(deg)
        h = jnp.maximum(
            dinv * (u0_ref[...] + u1_ref[...]) + dinv * dinv * xw1_ref[...]
            + b_ref[...], 0.0)
        xw2 = jnp.dot(h, w_ref[...], preferred_element_type=jnp.float32, precision=lax.Precision.HIGHEST)
        xw2_ref[...] = xw2
        y2_ref[...] = xw2 * dinv

    return pl.pallas_call(
        body,
        compiler_params=pltpu.CompilerParams(vmem_limit_bytes=100 * 1024 * 1024),
        out_shape=[
            jax.ShapeDtypeStruct((N, H), jnp.float32),
            jax.ShapeDtypeStruct((N, H), jnp.float32),
        ],
    )(U0, U1, xw1, d0, d1, b1, W2)


def _tc_last(U0, U1, xw2, d0, d1, b2, batch2, Wf1, bf1, Wf2, bf2):
    def body(u0_ref, u1_ref, xw2_ref, d0_ref, d1_ref, b_ref, bat_ref,
             wf1_ref, bf1_ref, wf2_ref, bf2_ref, out_ref):
        deg = d0_ref[...] + d1_ref[...] + 1.0
        dinv = lax.rsqrt(deg)
        h = jnp.maximum(
            dinv * (u0_ref[...] + u1_ref[...]) + dinv * dinv * xw2_ref[...]
            + b_ref[...], 0.0)
        gids = lax.broadcasted_iota(jnp.int32, (1, G), 1)
        onehot = (bat_ref[...] == gids).astype(jnp.float32)
        dn = (((0,), (0,)), ((), ()))
        sums = lax.dot_general(onehot, h, dn, preferred_element_type=jnp.float32, precision=lax.Precision.HIGHEST)
        cnts = lax.dot_general(onehot, jnp.ones((N, 1), jnp.float32), dn,
                               preferred_element_type=jnp.float32, precision=lax.Precision.HIGHEST)
        p = sums / jnp.maximum(cnts, 1.0)
        q = jnp.maximum(
            jnp.dot(p, wf1_ref[...], preferred_element_type=jnp.float32, precision=lax.Precision.HIGHEST)
            + bf1_ref[...], 0.0)
        out_ref[...] = (
            jnp.dot(q, wf2_ref[...], preferred_element_type=jnp.float32, precision=lax.Precision.HIGHEST)
            + bf2_ref[...])

    return pl.pallas_call(
        body,
        compiler_params=pltpu.CompilerParams(vmem_limit_bytes=100 * 1024 * 1024),
        out_shape=jax.ShapeDtypeStruct((G, 1), jnp.float32),
    )(U0, U1, xw2, d0, d1, b2, batch2, Wf1, bf1, Wf2, bf2)


def kernel(x, edge_index, edge_attr, batch, W1, b1, W2, b2, Wf1, bf1, Wf2, bf2):
    del edge_attr
    row = edge_index[0]
    col = edge_index[1]
    ones128 = jnp.ones((CHUNK,), jnp.float32)
    zslab = jnp.zeros((SLAB, H), jnp.float32)
    zcol = jnp.zeros((SLICE,), jnp.float32)

    xw1 = _tc_mm(x, W1)
    dp0, dp1 = _sc_deg(col.reshape(NCHUNKS, CHUNK), ones128, zcol)
    d0 = dp0.reshape(N, 1)
    d1 = dp1.reshape(N, 1)

    y1 = _tc_scale(xw1, d0, d1)
    ec = jnp.stack([row.reshape(NCHUNKS, CHUNK), col.reshape(NCHUNKS, CHUNK)],
                   axis=1)
    U10, U11 = _sc_agg(y1, ec, zslab)
    xw2, y2 = _tc_mid(U10, U11, xw1, d0, d1, b1.reshape(1, H), W2)
    U20, U21 = _sc_agg(y2, ec, zslab)
    out = _tc_last(U20, U21, xw2, d0, d1, b2.reshape(1, H),
                   batch.reshape(N, 1), Wf1, bf1.reshape(1, H // 2),
                   Wf2, bf2.reshape(1, 1))
    return out
